# baseline (speedup 1.0000x reference)
import numpy as np
import jax
import jax.numpy as jnp
from jax.experimental import pallas as pl
from jax.experimental.pallas import tpu as pltpu

LANES = 128
BL = 1024
OP_DT = jnp.bfloat16


def _conv1_tabs():
    e1t = np.zeros((256, 128), np.float32)
    for p in range(2):
        for o in range(6):
            for jp in range(12):
                e1t[p * 128 + o * 12 + jp, o] = 1.0
    c1 = np.zeros((2, 5, 144, 25), np.float32)
    m1 = np.zeros((2, 5, 256, 144), np.float32)
    for v, shift in ((0, 0), (1, 4)):
        for dj in range(5):
            for di in range(5):
                for jj in range(28):
                    k = di * 28 + jj + shift
                    c1[v, dj, k, di * 5 + dj] = 1.0
            for p in range(2):
                for o in range(6):
                    for jp in range(12):
                        n = p * 128 + o * 12 + jp
                        jj = 2 * jp + p + dj
                        for di in range(5):
                            m1[v, dj, n, di * 28 + jj + shift] = 1.0
    return e1t, c1, m1


def _conv2_tabs():
    e2t = np.zeros((256, 128), np.float32)
    for p in range(2):
        for o2 in range(12):
            for j2p in range(4):
                e2t[p * 128 + o2 * 4 + j2p, o2] = 1.0
    c2 = np.zeros((5, 640, 150), np.float32)
    m2 = np.zeros((5, 256, 640), np.float32)
    for dj in range(5):
        for di in range(5):
            for c in range(6):
                for ww in range(12):
                    c2[dj, di * 128 + c * 12 + ww, (di * 5 + dj) * 6 + c] = 1.0
        for p in range(2):
            for o2 in range(12):
                for j2p in range(4):
                    n = p * 128 + o2 * 4 + j2p
                    ww = 2 * j2p + p + dj
                    for di in range(5):
                        for c in range(6):
                            m2[dj, n, di * 128 + c * 12 + ww] = 1.0
    return e2t, c2, m2


def _fc1_tab():
    pf = np.zeros((512, 192), np.float32)
    for h in range(4):
        for c in range(12):
            for w in range(4):
                pf[h * 128 + c * 4 + w, h * 48 + w * 12 + c] = 1.0
    return pf


_E1T, _C1, _M1 = _conv1_tabs()
_E2T, _C2, _M2 = _conv2_tabs()
_PF = _fc1_tab()
_I128 = np.eye(128, dtype=np.float32)


def _dgt(a, b):
    return jax.lax.dot_general(a, b, (((1,), (1,)), ((), ())),
                               preferred_element_type=jnp.float32)


def _prep_kernel(c1w, c1b, c2w, c2b, fc1w, fc2w, outw, fc1b, fc2b, outb,
                 e1t, c1t, m1t, e2t, c2t, m2t, pf, i128,
                 w1e_o, w1o_o, b1_o, w2_o, b2_o, wf_o, bf_o, wg_o, bg_o,
                 wo_o, bo_o):
    v1t = _dgt(e1t[...], c1w[...])
    for v, ref in ((0, w1e_o), (1, w1o_o)):
        w = jnp.zeros((256, 144), jnp.float32)
        for dj in range(5):
            w = w + _dgt(v1t, c1t[v, dj]) * m1t[v, dj]
        ref[...] = w.astype(ref.dtype)
    b1_o[...] = _dgt(e1t[...], c1b[...])[:LANES]

    v2t = _dgt(e2t[...], c2w[...])
    w2 = jnp.zeros((256, 640), jnp.float32)
    for dj in range(5):
        w2 = w2 + _dgt(v2t, c2t[dj]) * m2t[dj]
    w2_o[...] = w2.astype(w2_o.dtype)
    b2_o[...] = _dgt(e2t[...], c2b[...])[:LANES]

    wf_o[...] = jax.lax.dot_general(
        fc1w[...], pf[...], (((0,), (1,)), ((), ())),
        preferred_element_type=jnp.float32).astype(wf_o.dtype)
    bf_o[...] = _dgt(i128[...], fc1b[...])
    wg_o[...] = fc2w[...].astype(wg_o.dtype)
    bg_o[...] = _dgt(i128[...], fc2b[...])
    wo_o[...] = outw[...].astype(wo_o.dtype)
    bo_o[...] = _dgt(i128[...], outb[...])


def _full(shape):
    return pl.BlockSpec(shape, lambda: (0,) * len(shape))


def _prep(c1_w, c1_b, c2_w, c2_b, fc1_w, fc2_w, out_w, fc1_b, fc2_b, out_b):
    f32 = jnp.float32
    return pl.pallas_call(
        _prep_kernel,
        out_shape=[jax.ShapeDtypeStruct((256, 144), OP_DT),
                   jax.ShapeDtypeStruct((256, 144), OP_DT),
                   jax.ShapeDtypeStruct((128, 1), f32),
                   jax.ShapeDtypeStruct((256, 640), OP_DT),
                   jax.ShapeDtypeStruct((128, 1), f32),
                   jax.ShapeDtypeStruct((128, 512), OP_DT),
                   jax.ShapeDtypeStruct((128, 1), f32),
                   jax.ShapeDtypeStruct((128, 128), OP_DT),
                   jax.ShapeDtypeStruct((128, 1), f32),
                   jax.ShapeDtypeStruct((128, 128), OP_DT),
                   jax.ShapeDtypeStruct((128, 1), f32)],
        in_specs=[_full((25, 128)), _full((1, 128)),
                  _full((150, 128)), _full((1, 128)),
                  _full((192, 128)), _full((128, 128)), _full((128, 128)),
                  _full((1, 128)), _full((1, 128)), _full((1, 128)),
                  _full((256, 128)), _full((2, 5, 144, 25)),
                  _full((2, 5, 256, 144)),
                  _full((256, 128)), _full((5, 640, 150)),
                  _full((5, 256, 640)),
                  _full((512, 192)), _full((128, 128))],
        out_specs=[_full((256, 144)), _full((256, 144)), _full((128, 1)),
                   _full((256, 640)), _full((128, 1)),
                   _full((128, 512)), _full((128, 1)),
                   _full((128, 128)), _full((128, 1)),
                   _full((128, 128)), _full((128, 1))],
    )(c1_w, c1_b, c2_w, c2_b, fc1_w, fc2_w, out_w, fc1_b, fc2_b, out_b,
      _E1T, _C1, _M1, _E2T, _C2, _M2, _PF, _I128)


def _dot(a, b):
    return jnp.dot(a, b, preferred_element_type=jnp.float32)


def _dga(a, b):
    return jax.lax.dot_general(a, b, (((0,), (0,)), ((), ())),
                               preferred_element_type=jnp.float32)


def _lenet_kernel(x_ref, w1e_ref, w1o_ref, b1_ref, w2_ref, b2_ref,
                  wf_ref, bf_ref, wg_ref, bg_ref, wo_ref, bo_ref,
                  o_ref, s1, s2):
    xb = x_ref[...].astype(OP_DT)
    for r in range(12):
        m = None
        for i in (2 * r, 2 * r + 1):
            if i % 2 == 0:
                d = _dot(w1e_ref[...], xb[i * 28:i * 28 + 144])
            else:
                d = _dot(w1o_ref[...], xb[i * 28 - 4:i * 28 + 140])
            mm = jnp.maximum(d[:LANES], d[LANES:])
            m = mm if m is None else jnp.maximum(m, mm)
        s1[r * LANES:(r + 1) * LANES, :] = (
            jnp.maximum(m + b1_ref[...], 0.0).astype(OP_DT))

    for r in range(4):
        m = None
        for i in (2 * r, 2 * r + 1):
            d = _dot(w2_ref[...], s1[i * LANES:i * LANES + 640, :])
            mm = jnp.maximum(d[:LANES], d[LANES:])
            m = mm if m is None else jnp.maximum(m, mm)
        s2[r * LANES:(r + 1) * LANES, :] = (
            jnp.maximum(m + b2_ref[...], 0.0).astype(OP_DT))

    h = jnp.maximum(_dot(wf_ref[...], s2[...]) + bf_ref[...], 0.0)
    h = h.astype(OP_DT)
    h = jnp.maximum(_dga(wg_ref[...], h) + bg_ref[...], 0.0)
    h = h.astype(OP_DT)
    o = _dga(wo_ref[...], h) + bo_ref[...]
    o_ref[...] = o[:10, :]


def kernel(c1_w, c1_b, c2_w, c2_b, fc1_w, fc1_b, fc2_w, fc2_b, out_w, out_b, x):
    B = x.shape[0]
    bl = BL if B % BL == 0 else B
    xt = jnp.transpose(x, (2, 3, 1, 0)).reshape(784, B)

    (w1e, w1o, b1, w2, b2, wf, bf, wg, bg, wo, bo) = _prep(
        c1_w, c1_b, c2_w, c2_b, fc1_w, fc2_w, out_w, fc1_b, fc2_b, out_b)

    out_t = pl.pallas_call(
        _lenet_kernel,
        out_shape=jax.ShapeDtypeStruct((10, B), jnp.float32),
        grid=(B // bl,),
        in_specs=[pl.BlockSpec((784, bl), lambda i: (0, i)),
                  pl.BlockSpec(w1e.shape, lambda i: (0, 0)),
                  pl.BlockSpec(w1o.shape, lambda i: (0, 0)),
                  pl.BlockSpec(b1.shape, lambda i: (0, 0)),
                  pl.BlockSpec(w2.shape, lambda i: (0, 0)),
                  pl.BlockSpec(b2.shape, lambda i: (0, 0)),
                  pl.BlockSpec(wf.shape, lambda i: (0, 0)),
                  pl.BlockSpec(bf.shape, lambda i: (0, 0)),
                  pl.BlockSpec(wg.shape, lambda i: (0, 0)),
                  pl.BlockSpec(bg.shape, lambda i: (0, 0)),
                  pl.BlockSpec(wo.shape, lambda i: (0, 0)),
                  pl.BlockSpec(bo.shape, lambda i: (0, 0))],
        out_specs=pl.BlockSpec((10, bl), lambda i: (0, i)),
        scratch_shapes=[pltpu.VMEM((12 * LANES, bl), OP_DT),
                        pltpu.VMEM((4 * LANES, bl), OP_DT)],
        compiler_params=pltpu.CompilerParams(
            dimension_semantics=("parallel",)),
    )(xt, w1e, w1o, b1, w2, b2, wf, bf, wg, bg, wo, bo)
    return out_t.T


# speedup vs baseline: 2.3914x; 2.3914x over previous
import numpy as np
import jax
import jax.numpy as jnp
from jax.experimental import pallas as pl
from jax.experimental.pallas import tpu as pltpu

LANES = 128
BL = 1024
OP_DT = jnp.bfloat16


def _conv1_tabs():
    e1t = np.zeros((256, 128), np.float32)
    for p in range(2):
        for o in range(6):
            for jp in range(12):
                e1t[p * 128 + o * 12 + jp, o] = 1.0
    c1 = np.zeros((2, 5, 144, 25), np.float32)
    m1 = np.zeros((2, 5, 256, 144), np.float32)
    for v, shift in ((0, 0), (1, 4)):
        for dj in range(5):
            for di in range(5):
                for jj in range(28):
                    k = di * 28 + jj + shift
                    c1[v, dj, k, di * 5 + dj] = 1.0
            for p in range(2):
                for o in range(6):
                    for jp in range(12):
                        n = p * 128 + o * 12 + jp
                        jj = 2 * jp + p + dj
                        for di in range(5):
                            m1[v, dj, n, di * 28 + jj + shift] = 1.0
    return e1t, c1, m1


def _conv2_tabs():
    e2t = np.zeros((256, 128), np.float32)
    for p in range(2):
        for o2 in range(12):
            for j2p in range(4):
                e2t[p * 128 + o2 * 4 + j2p, o2] = 1.0
    c2 = np.zeros((5, 640, 150), np.float32)
    m2 = np.zeros((5, 256, 640), np.float32)
    for dj in range(5):
        for di in range(5):
            for c in range(6):
                for ww in range(12):
                    c2[dj, di * 128 + c * 12 + ww, (di * 5 + dj) * 6 + c] = 1.0
        for p in range(2):
            for o2 in range(12):
                for j2p in range(4):
                    n = p * 128 + o2 * 4 + j2p
                    ww = 2 * j2p + p + dj
                    for di in range(5):
                        for c in range(6):
                            m2[dj, n, di * 128 + c * 12 + ww] = 1.0
    return e2t, c2, m2


def _fc1_tab():
    pf = np.zeros((512, 192), np.float32)
    for h in range(4):
        for c in range(12):
            for w in range(4):
                pf[h * 128 + c * 4 + w, h * 48 + w * 12 + c] = 1.0
    return pf


_E1T, _C1, _M1 = _conv1_tabs()
_E2T, _C2, _M2 = _conv2_tabs()
_PF = _fc1_tab()
_I128 = np.eye(128, dtype=np.float32)


def _dgt(a, b):
    return jax.lax.dot_general(a, b, (((1,), (1,)), ((), ())),
                               preferred_element_type=jnp.float32)


def _prep_kernel(c1w, c1b, c2w, c2b, fc1w, fc2w, outw, fc1b, fc2b, outb,
                 e1t, c1t, m1t, e2t, c2t, m2t, pf, i128,
                 w1e_o, w1o_o, b1_o, w2_o, b2_o, wf_o, bf_o, wg_o, bg_o,
                 wo_o, bo_o):
    v1t = _dgt(e1t[...], c1w[...])
    for v, ref in ((0, w1e_o), (1, w1o_o)):
        w = jnp.zeros((256, 144), jnp.float32)
        for dj in range(5):
            w = w + _dgt(v1t, c1t[v, dj]) * m1t[v, dj]
        ref[...] = w.astype(ref.dtype)
    b1_o[...] = _dgt(e1t[...], c1b[...])[:LANES]

    v2t = _dgt(e2t[...], c2w[...])
    w2 = jnp.zeros((256, 640), jnp.float32)
    for dj in range(5):
        w2 = w2 + _dgt(v2t, c2t[dj]) * m2t[dj]
    w2_o[...] = w2.astype(w2_o.dtype)
    b2_o[...] = _dgt(e2t[...], c2b[...])[:LANES]

    wf_o[...] = jax.lax.dot_general(
        fc1w[...], pf[...], (((0,), (1,)), ((), ())),
        preferred_element_type=jnp.float32).astype(wf_o.dtype)
    bf_o[...] = _dgt(i128[...], fc1b[...])
    wg_o[...] = fc2w[...].astype(wg_o.dtype)
    bg_o[...] = _dgt(i128[...], fc2b[...])
    wo_o[...] = outw[...].astype(wo_o.dtype)
    bo_o[...] = _dgt(i128[...], outb[...])


def _full(shape):
    return pl.BlockSpec(shape, lambda: (0,) * len(shape))


def _prep(c1_w, c1_b, c2_w, c2_b, fc1_w, fc2_w, out_w, fc1_b, fc2_b, out_b):
    f32 = jnp.float32
    return pl.pallas_call(
        _prep_kernel,
        out_shape=[jax.ShapeDtypeStruct((256, 144), OP_DT),
                   jax.ShapeDtypeStruct((256, 144), OP_DT),
                   jax.ShapeDtypeStruct((128, 1), f32),
                   jax.ShapeDtypeStruct((256, 640), OP_DT),
                   jax.ShapeDtypeStruct((128, 1), f32),
                   jax.ShapeDtypeStruct((128, 512), OP_DT),
                   jax.ShapeDtypeStruct((128, 1), f32),
                   jax.ShapeDtypeStruct((128, 128), OP_DT),
                   jax.ShapeDtypeStruct((128, 1), f32),
                   jax.ShapeDtypeStruct((128, 128), OP_DT),
                   jax.ShapeDtypeStruct((128, 1), f32)],
        in_specs=[_full((25, 128)), _full((1, 128)),
                  _full((150, 128)), _full((1, 128)),
                  _full((192, 128)), _full((128, 128)), _full((128, 128)),
                  _full((1, 128)), _full((1, 128)), _full((1, 128)),
                  _full((256, 128)), _full((2, 5, 144, 25)),
                  _full((2, 5, 256, 144)),
                  _full((256, 128)), _full((5, 640, 150)),
                  _full((5, 256, 640)),
                  _full((512, 192)), _full((128, 128))],
        out_specs=[_full((256, 144)), _full((256, 144)), _full((128, 1)),
                   _full((256, 640)), _full((128, 1)),
                   _full((128, 512)), _full((128, 1)),
                   _full((128, 128)), _full((128, 1)),
                   _full((128, 128)), _full((128, 1))],
    )(c1_w, c1_b, c2_w, c2_b, fc1_w, fc2_w, out_w, fc1_b, fc2_b, out_b,
      _E1T, _C1, _M1, _E2T, _C2, _M2, _PF, _I128)


def _dot(a, b):
    return jnp.dot(a, b, preferred_element_type=jnp.float32)


def _dga(a, b):
    return jax.lax.dot_general(a, b, (((0,), (0,)), ((), ())),
                               preferred_element_type=jnp.float32)


def _lenet_kernel(x_ref, w1e_ref, w1o_ref, b1_ref, w2_ref, b2_ref,
                  wf_ref, bf_ref, wg_ref, bg_ref, wo_ref, bo_ref,
                  o_ref, s1, s2):
    xb = x_ref[...].astype(OP_DT).reshape(784, -1)
    for r in range(12):
        m = None
        for i in (2 * r, 2 * r + 1):
            if i % 2 == 0:
                d = _dot(w1e_ref[...], xb[i * 28:i * 28 + 144])
            else:
                d = _dot(w1o_ref[...], xb[i * 28 - 4:i * 28 + 140])
            mm = jnp.maximum(d[:LANES], d[LANES:])
            m = mm if m is None else jnp.maximum(m, mm)
        s1[r * LANES:(r + 1) * LANES, :] = (
            jnp.maximum(m + b1_ref[...], 0.0).astype(OP_DT))

    for r in range(4):
        m = None
        for i in (2 * r, 2 * r + 1):
            d = _dot(w2_ref[...], s1[i * LANES:i * LANES + 640, :])
            mm = jnp.maximum(d[:LANES], d[LANES:])
            m = mm if m is None else jnp.maximum(m, mm)
        s2[r * LANES:(r + 1) * LANES, :] = (
            jnp.maximum(m + b2_ref[...], 0.0).astype(OP_DT))

    h = jnp.maximum(_dot(wf_ref[...], s2[...]) + bf_ref[...], 0.0)
    h = h.astype(OP_DT)
    h = jnp.maximum(_dga(wg_ref[...], h) + bg_ref[...], 0.0)
    h = h.astype(OP_DT)
    o = _dga(wo_ref[...], h) + bo_ref[...]
    o_ref[...] = o[:10, :]


def kernel(c1_w, c1_b, c2_w, c2_b, fc1_w, fc1_b, fc2_w, fc2_b, out_w, out_b, x):
    B = x.shape[0]
    bl = BL if B % BL == 0 else B
    xt = jnp.transpose(x, (2, 3, 1, 0)).reshape(784, B // 128, 128)

    (w1e, w1o, b1, w2, b2, wf, bf, wg, bg, wo, bo) = _prep(
        c1_w, c1_b, c2_w, c2_b, fc1_w, fc2_w, out_w, fc1_b, fc2_b, out_b)

    out_t = pl.pallas_call(
        _lenet_kernel,
        out_shape=jax.ShapeDtypeStruct((10, B), jnp.float32),
        grid=(B // bl,),
        in_specs=[pl.BlockSpec((784, bl // 128, 128), lambda i: (0, i, 0)),
                  pl.BlockSpec(w1e.shape, lambda i: (0, 0)),
                  pl.BlockSpec(w1o.shape, lambda i: (0, 0)),
                  pl.BlockSpec(b1.shape, lambda i: (0, 0)),
                  pl.BlockSpec(w2.shape, lambda i: (0, 0)),
                  pl.BlockSpec(b2.shape, lambda i: (0, 0)),
                  pl.BlockSpec(wf.shape, lambda i: (0, 0)),
                  pl.BlockSpec(bf.shape, lambda i: (0, 0)),
                  pl.BlockSpec(wg.shape, lambda i: (0, 0)),
                  pl.BlockSpec(bg.shape, lambda i: (0, 0)),
                  pl.BlockSpec(wo.shape, lambda i: (0, 0)),
                  pl.BlockSpec(bo.shape, lambda i: (0, 0))],
        out_specs=pl.BlockSpec((10, bl), lambda i: (0, i)),
        scratch_shapes=[pltpu.VMEM((12 * LANES, bl), OP_DT),
                        pltpu.VMEM((4 * LANES, bl), OP_DT)],
        compiler_params=pltpu.CompilerParams(
            dimension_semantics=("parallel",)),
    )(xt, w1e, w1o, b1, w2, b2, wf, bf, wg, bg, wo, bo)
    return out_t.T


# speedup vs baseline: 4.7853x; 2.0011x over previous
import numpy as np
import jax
import jax.numpy as jnp
from jax.experimental import pallas as pl
from jax.experimental.pallas import tpu as pltpu

LANES = 128
BL = 1024
OP_DT = jnp.bfloat16


def _conv1_tabs():
    e1t = np.zeros((144, 128), np.float32)
    for p in range(2):
        for o in range(6):
            for jp in range(12):
                e1t[p * 72 + o * 12 + jp, o] = 1.0
    c1 = np.zeros((5, 140, 25), np.float32)
    m1 = np.zeros((5, 144, 140), np.float32)
    for dj in range(5):
        for di in range(5):
            for jj in range(28):
                c1[dj, di * 28 + jj, di * 5 + dj] = 1.0
        for p in range(2):
            for o in range(6):
                for jp in range(12):
                    n = p * 72 + o * 12 + jp
                    jj = 2 * jp + p + dj
                    for di in range(5):
                        m1[dj, n, di * 28 + jj] = 1.0
    return e1t, c1, m1


def _conv2_tabs():
    e2t = np.zeros((96, 128), np.float32)
    for p in range(2):
        for o2 in range(12):
            for j2p in range(4):
                e2t[p * 48 + o2 * 4 + j2p, o2] = 1.0
    c2 = np.zeros((5, 360, 150), np.float32)
    m2 = np.zeros((5, 96, 360), np.float32)
    for dj in range(5):
        for di in range(5):
            for c in range(6):
                for ww in range(12):
                    c2[dj, di * 72 + c * 12 + ww, (di * 5 + dj) * 6 + c] = 1.0
        for p in range(2):
            for o2 in range(12):
                for j2p in range(4):
                    n = p * 48 + o2 * 4 + j2p
                    ww = 2 * j2p + p + dj
                    for di in range(5):
                        for c in range(6):
                            m2[dj, n, di * 72 + c * 12 + ww] = 1.0
    return e2t, c2, m2


def _fc1_tab():
    pf = np.zeros((192, 192), np.float32)
    for h in range(4):
        for c in range(12):
            for w in range(4):
                pf[h * 48 + c * 4 + w, h * 48 + w * 12 + c] = 1.0
    return pf


_E1T, _C1, _M1 = _conv1_tabs()
_E2T, _C2, _M2 = _conv2_tabs()
_PF = _fc1_tab()
_I128 = np.eye(128, dtype=np.float32)


def _dgt(a, b):
    return jax.lax.dot_general(a, b, (((1,), (1,)), ((), ())),
                               preferred_element_type=jnp.float32)


def _prep_kernel(c1w, c1b, c2w, c2b, fc1w, fc2w, outw, fc1b, fc2b, outb,
                 e1t, c1t, m1t, e2t, c2t, m2t, pf, i128,
                 w1_o, b1_o, w2_o, b2_o, wf_o, bf_o, wg_o, bg_o,
                 wo_o, bo_o):
    v1t = _dgt(e1t[...], c1w[...])
    w1 = jnp.zeros((144, 140), jnp.float32)
    for dj in range(5):
        w1 = w1 + _dgt(v1t, c1t[dj]) * m1t[dj]
    w1_o[...] = jnp.concatenate(
        [jnp.pad(w1, ((0, 0), (0, 28))), jnp.pad(w1, ((0, 0), (28, 0)))],
        axis=0).astype(w1_o.dtype)
    b1_o[...] = _dgt(e1t[...], c1b[...])[:72]

    v2t = _dgt(e2t[...], c2w[...])
    w2 = jnp.zeros((96, 360), jnp.float32)
    for dj in range(5):
        w2 = w2 + _dgt(v2t, c2t[dj]) * m2t[dj]
    w2_o[...] = jnp.concatenate(
        [jnp.pad(w2, ((0, 0), (0, 72))), jnp.pad(w2, ((0, 0), (72, 0)))],
        axis=0).astype(w2_o.dtype)
    b2_o[...] = _dgt(e2t[...], c2b[...])[:48]

    wf_o[...] = jax.lax.dot_general(
        fc1w[...], pf[...], (((0,), (1,)), ((), ())),
        preferred_element_type=jnp.float32).astype(wf_o.dtype)
    bf_o[...] = _dgt(i128[...], fc1b[...])
    wg_o[...] = fc2w[...].astype(wg_o.dtype)
    bg_o[...] = _dgt(i128[...], fc2b[...])
    wo_o[...] = outw[...].astype(wo_o.dtype)
    bo_o[...] = _dgt(i128[...], outb[...])


def _full(shape):
    return pl.BlockSpec(shape, lambda: (0,) * len(shape))


def _prep(c1_w, c1_b, c2_w, c2_b, fc1_w, fc2_w, out_w, fc1_b, fc2_b, out_b):
    f32 = jnp.float32
    return pl.pallas_call(
        _prep_kernel,
        out_shape=[jax.ShapeDtypeStruct((288, 168), OP_DT),
                   jax.ShapeDtypeStruct((72, 1), f32),
                   jax.ShapeDtypeStruct((192, 432), OP_DT),
                   jax.ShapeDtypeStruct((48, 1), f32),
                   jax.ShapeDtypeStruct((128, 192), OP_DT),
                   jax.ShapeDtypeStruct((128, 1), f32),
                   jax.ShapeDtypeStruct((128, 128), OP_DT),
                   jax.ShapeDtypeStruct((128, 1), f32),
                   jax.ShapeDtypeStruct((128, 128), OP_DT),
                   jax.ShapeDtypeStruct((128, 1), f32)],
        in_specs=[_full((25, 128)), _full((1, 128)),
                  _full((150, 128)), _full((1, 128)),
                  _full((192, 128)), _full((128, 128)), _full((128, 128)),
                  _full((1, 128)), _full((1, 128)), _full((1, 128)),
                  _full((144, 128)), _full((5, 140, 25)),
                  _full((5, 144, 140)),
                  _full((96, 128)), _full((5, 360, 150)),
                  _full((5, 96, 360)),
                  _full((192, 192)), _full((128, 128))],
        out_specs=[_full((288, 168)), _full((72, 1)),
                   _full((192, 432)), _full((48, 1)),
                   _full((128, 192)), _full((128, 1)),
                   _full((128, 128)), _full((128, 1)),
                   _full((128, 128)), _full((128, 1))],
    )(c1_w, c1_b, c2_w, c2_b, fc1_w, fc2_w, out_w, fc1_b, fc2_b, out_b,
      _E1T, _C1, _M1, _E2T, _C2, _M2, _PF, _I128)


def _dot(a, b):
    return jnp.dot(a, b, preferred_element_type=jnp.float32)


def _dga(a, b):
    return jax.lax.dot_general(a, b, (((0,), (0,)), ((), ())),
                               preferred_element_type=jnp.float32)


def _lenet_kernel(x_ref, w1_ref, b1_ref, w2_ref, b2_ref,
                  wf_ref, bf_ref, wg_ref, bg_ref, wo_ref, bo_ref,
                  o_ref, s1, s2):
    xb = x_ref[...].astype(OP_DT).reshape(784, -1)
    for r in range(12):
        d = _dot(w1_ref[...], xb[r * 56:r * 56 + 168])
        m = jnp.maximum(jnp.maximum(d[:72], d[72:144]),
                        jnp.maximum(d[144:216], d[216:288]))
        s1[r * 72:(r + 1) * 72, :] = (
            jnp.maximum(m + b1_ref[...], 0.0).astype(OP_DT))

    for r in range(4):
        d = _dot(w2_ref[...], s1[r * 144:r * 144 + 432, :])
        m = jnp.maximum(jnp.maximum(d[:48], d[48:96]),
                        jnp.maximum(d[96:144], d[144:192]))
        s2[r * 48:(r + 1) * 48, :] = (
            jnp.maximum(m + b2_ref[...], 0.0).astype(OP_DT))

    h = jnp.maximum(_dot(wf_ref[...], s2[...]) + bf_ref[...], 0.0)
    h = h.astype(OP_DT)
    h = jnp.maximum(_dga(wg_ref[...], h) + bg_ref[...], 0.0)
    h = h.astype(OP_DT)
    o = _dga(wo_ref[...], h) + bo_ref[...]
    o_ref[...] = o[:10, :]


def kernel(c1_w, c1_b, c2_w, c2_b, fc1_w, fc1_b, fc2_w, fc2_b, out_w, out_b, x):
    B = x.shape[0]
    bl = BL if B % BL == 0 else B
    xt = jnp.transpose(x, (2, 3, 1, 0)).reshape(784, B // 128, 128)

    (w1, b1, w2, b2, wf, bf, wg, bg, wo, bo) = _prep(
        c1_w, c1_b, c2_w, c2_b, fc1_w, fc2_w, out_w, fc1_b, fc2_b, out_b)

    out_t = pl.pallas_call(
        _lenet_kernel,
        out_shape=jax.ShapeDtypeStruct((10, B), jnp.float32),
        grid=(B // bl,),
        in_specs=[pl.BlockSpec((784, bl // 128, 128), lambda i: (0, i, 0)),
                  pl.BlockSpec(w1.shape, lambda i: (0, 0)),
                  pl.BlockSpec(b1.shape, lambda i: (0, 0)),
                  pl.BlockSpec(w2.shape, lambda i: (0, 0)),
                  pl.BlockSpec(b2.shape, lambda i: (0, 0)),
                  pl.BlockSpec(wf.shape, lambda i: (0, 0)),
                  pl.BlockSpec(bf.shape, lambda i: (0, 0)),
                  pl.BlockSpec(wg.shape, lambda i: (0, 0)),
                  pl.BlockSpec(bg.shape, lambda i: (0, 0)),
                  pl.BlockSpec(wo.shape, lambda i: (0, 0)),
                  pl.BlockSpec(bo.shape, lambda i: (0, 0))],
        out_specs=pl.BlockSpec((10, bl), lambda i: (0, i)),
        scratch_shapes=[pltpu.VMEM((12 * 72, bl), OP_DT),
                        pltpu.VMEM((4 * 48, bl), OP_DT)],
        compiler_params=pltpu.CompilerParams(
            dimension_semantics=("parallel",)),
    )(xt, w1, b1, w2, b2, wf, bf, wg, bg, wo, bo)
    return out_t.T


# speedup vs baseline: 5.0342x; 1.0520x over previous
import numpy as np
import jax
import jax.numpy as jnp
from jax.experimental import pallas as pl
from jax.experimental.pallas import tpu as pltpu

LANES = 128
BL = 2048
OP_DT = jnp.bfloat16


def _conv1_tabs():
    e1t = np.zeros((144, 128), np.float32)
    for p in range(2):
        for o in range(6):
            for jp in range(12):
                e1t[p * 72 + o * 12 + jp, o] = 1.0
    c1 = np.zeros((5, 140, 25), np.float32)
    m1 = np.zeros((5, 144, 140), np.float32)
    for dj in range(5):
        for di in range(5):
            for jj in range(28):
                c1[dj, di * 28 + jj, di * 5 + dj] = 1.0
        for p in range(2):
            for o in range(6):
                for jp in range(12):
                    n = p * 72 + o * 12 + jp
                    jj = 2 * jp + p + dj
                    for di in range(5):
                        m1[dj, n, di * 28 + jj] = 1.0
    return e1t, c1, m1


def _conv2_tabs():
    e2t = np.zeros((96, 128), np.float32)
    for p in range(2):
        for o2 in range(12):
            for j2p in range(4):
                e2t[p * 48 + o2 * 4 + j2p, o2] = 1.0
    c2 = np.zeros((5, 360, 150), np.float32)
    m2 = np.zeros((5, 96, 360), np.float32)
    for dj in range(5):
        for di in range(5):
            for c in range(6):
                for ww in range(12):
                    c2[dj, di * 72 + c * 12 + ww, (di * 5 + dj) * 6 + c] = 1.0
        for p in range(2):
            for o2 in range(12):
                for j2p in range(4):
                    n = p * 48 + o2 * 4 + j2p
                    ww = 2 * j2p + p + dj
                    for di in range(5):
                        for c in range(6):
                            m2[dj, n, di * 72 + c * 12 + ww] = 1.0
    return e2t, c2, m2


def _fc1_tab():
    pf = np.zeros((192, 192), np.float32)
    for h in range(4):
        for c in range(12):
            for w in range(4):
                pf[h * 48 + c * 4 + w, h * 48 + w * 12 + c] = 1.0
    return pf


_E1T, _C1, _M1 = _conv1_tabs()
_E2T, _C2, _M2 = _conv2_tabs()
_PF = _fc1_tab()
_I128 = np.eye(128, dtype=np.float32)


def _dgt(a, b):
    return jax.lax.dot_general(a, b, (((1,), (1,)), ((), ())),
                               preferred_element_type=jnp.float32)


def _prep_kernel(c1w, c1b, c2w, c2b, fc1w, fc2w, outw, fc1b, fc2b, outb,
                 e1t, c1t, m1t, e2t, c2t, m2t, pf, i128,
                 w1_o, b1_o, w2_o, b2_o, wf_o, bf_o, wg_o, bg_o,
                 wo_o, bo_o):
    v1t = _dgt(e1t[...], c1w[...])
    w1 = jnp.zeros((144, 140), jnp.float32)
    for dj in range(5):
        w1 = w1 + _dgt(v1t, c1t[dj]) * m1t[dj]
    w1_o[...] = jnp.concatenate(
        [jnp.pad(w1, ((0, 0), (0, 28))), jnp.pad(w1, ((0, 0), (28, 0)))],
        axis=0).astype(w1_o.dtype)
    b1_o[...] = _dgt(e1t[...], c1b[...])[:72]

    v2t = _dgt(e2t[...], c2w[...])
    w2 = jnp.zeros((96, 360), jnp.float32)
    for dj in range(5):
        w2 = w2 + _dgt(v2t, c2t[dj]) * m2t[dj]
    w2_o[...] = jnp.concatenate(
        [jnp.pad(w2, ((0, 0), (0, 72))), jnp.pad(w2, ((0, 0), (72, 0)))],
        axis=0).astype(w2_o.dtype)
    b2_o[...] = _dgt(e2t[...], c2b[...])[:48]

    wf_o[...] = jax.lax.dot_general(
        fc1w[...], pf[...], (((0,), (1,)), ((), ())),
        preferred_element_type=jnp.float32).astype(wf_o.dtype)
    bf_o[...] = _dgt(i128[...], fc1b[...])
    wg_o[...] = fc2w[...].astype(wg_o.dtype)
    bg_o[...] = _dgt(i128[...], fc2b[...])
    wo_o[...] = outw[...].astype(wo_o.dtype)
    bo_o[...] = _dgt(i128[...], outb[...])


def _full(shape):
    return pl.BlockSpec(shape, lambda: (0,) * len(shape))


def _prep(c1_w, c1_b, c2_w, c2_b, fc1_w, fc2_w, out_w, fc1_b, fc2_b, out_b):
    f32 = jnp.float32
    return pl.pallas_call(
        _prep_kernel,
        out_shape=[jax.ShapeDtypeStruct((288, 168), OP_DT),
                   jax.ShapeDtypeStruct((72, 1), f32),
                   jax.ShapeDtypeStruct((192, 432), OP_DT),
                   jax.ShapeDtypeStruct((48, 1), f32),
                   jax.ShapeDtypeStruct((128, 192), OP_DT),
                   jax.ShapeDtypeStruct((128, 1), f32),
                   jax.ShapeDtypeStruct((128, 128), OP_DT),
                   jax.ShapeDtypeStruct((128, 1), f32),
                   jax.ShapeDtypeStruct((128, 128), OP_DT),
                   jax.ShapeDtypeStruct((128, 1), f32)],
        in_specs=[_full((25, 128)), _full((1, 128)),
                  _full((150, 128)), _full((1, 128)),
                  _full((192, 128)), _full((128, 128)), _full((128, 128)),
                  _full((1, 128)), _full((1, 128)), _full((1, 128)),
                  _full((144, 128)), _full((5, 140, 25)),
                  _full((5, 144, 140)),
                  _full((96, 128)), _full((5, 360, 150)),
                  _full((5, 96, 360)),
                  _full((192, 192)), _full((128, 128))],
        out_specs=[_full((288, 168)), _full((72, 1)),
                   _full((192, 432)), _full((48, 1)),
                   _full((128, 192)), _full((128, 1)),
                   _full((128, 128)), _full((128, 1)),
                   _full((128, 128)), _full((128, 1))],
    )(c1_w, c1_b, c2_w, c2_b, fc1_w, fc2_w, out_w, fc1_b, fc2_b, out_b,
      _E1T, _C1, _M1, _E2T, _C2, _M2, _PF, _I128)


def _dot(a, b):
    return jnp.dot(a, b, preferred_element_type=jnp.float32)


def _dga(a, b):
    return jax.lax.dot_general(a, b, (((0,), (0,)), ((), ())),
                               preferred_element_type=jnp.float32)


def _lenet_kernel(x_ref, w1_ref, b1_ref, w2_ref, b2_ref,
                  wf_ref, bf_ref, wg_ref, bg_ref, wo_ref, bo_ref,
                  o_ref, s1, s2):
    xb = x_ref[...].astype(OP_DT).reshape(784, -1)
    for r in range(12):
        d = _dot(w1_ref[...], xb[r * 56:r * 56 + 168])
        m = jnp.maximum(jnp.maximum(d[:72], d[72:144]),
                        jnp.maximum(d[144:216], d[216:288]))
        s1[r * 72:(r + 1) * 72, :] = (
            jnp.maximum(m + b1_ref[...], 0.0).astype(OP_DT))

    for r in range(4):
        d = _dot(w2_ref[...], s1[r * 144:r * 144 + 432, :])
        m = jnp.maximum(jnp.maximum(d[:48], d[48:96]),
                        jnp.maximum(d[96:144], d[144:192]))
        s2[r * 48:(r + 1) * 48, :] = (
            jnp.maximum(m + b2_ref[...], 0.0).astype(OP_DT))

    h = jnp.maximum(_dot(wf_ref[...], s2[...]) + bf_ref[...], 0.0)
    h = h.astype(OP_DT)
    h = jnp.maximum(_dga(wg_ref[...], h) + bg_ref[...], 0.0)
    h = h.astype(OP_DT)
    o = _dga(wo_ref[...], h) + bo_ref[...]
    o_ref[...] = o[:10, :]


def kernel(c1_w, c1_b, c2_w, c2_b, fc1_w, fc1_b, fc2_w, fc2_b, out_w, out_b, x):
    B = x.shape[0]
    bl = BL if B % BL == 0 else B
    xt = jnp.transpose(x, (2, 3, 1, 0)).reshape(784, B // 128, 128)

    (w1, b1, w2, b2, wf, bf, wg, bg, wo, bo) = _prep(
        c1_w, c1_b, c2_w, c2_b, fc1_w, fc2_w, out_w, fc1_b, fc2_b, out_b)

    out_t = pl.pallas_call(
        _lenet_kernel,
        out_shape=jax.ShapeDtypeStruct((10, B), jnp.float32),
        grid=(B // bl,),
        in_specs=[pl.BlockSpec((784, bl // 128, 128), lambda i: (0, i, 0)),
                  pl.BlockSpec(w1.shape, lambda i: (0, 0)),
                  pl.BlockSpec(b1.shape, lambda i: (0, 0)),
                  pl.BlockSpec(w2.shape, lambda i: (0, 0)),
                  pl.BlockSpec(b2.shape, lambda i: (0, 0)),
                  pl.BlockSpec(wf.shape, lambda i: (0, 0)),
                  pl.BlockSpec(bf.shape, lambda i: (0, 0)),
                  pl.BlockSpec(wg.shape, lambda i: (0, 0)),
                  pl.BlockSpec(bg.shape, lambda i: (0, 0)),
                  pl.BlockSpec(wo.shape, lambda i: (0, 0)),
                  pl.BlockSpec(bo.shape, lambda i: (0, 0))],
        out_specs=pl.BlockSpec((10, bl), lambda i: (0, i)),
        scratch_shapes=[pltpu.VMEM((12 * 72, bl), OP_DT),
                        pltpu.VMEM((4 * 48, bl), OP_DT)],
        compiler_params=pltpu.CompilerParams(
            dimension_semantics=("parallel",)),
    )(xt, w1, b1, w2, b2, wf, bf, wg, bg, wo, bo)
    return out_t.T


# speedup vs baseline: 5.0419x; 1.0015x over previous
import numpy as np
import jax
import jax.numpy as jnp
from jax.experimental import pallas as pl
from jax.experimental.pallas import tpu as pltpu

LANES = 128
BL = 2048
OP_DT = jnp.bfloat16


def _conv1_tabs():
    e1t = np.zeros((144, 128), np.float32)
    for p in range(2):
        for o in range(6):
            for jp in range(12):
                e1t[p * 72 + o * 12 + jp, o] = 1.0
    c1 = np.zeros((5, 140, 25), np.float32)
    m1 = np.zeros((5, 144, 140), np.float32)
    for dj in range(5):
        for di in range(5):
            for jj in range(28):
                c1[dj, di * 28 + jj, di * 5 + dj] = 1.0
        for p in range(2):
            for o in range(6):
                for jp in range(12):
                    n = p * 72 + o * 12 + jp
                    jj = 2 * jp + p + dj
                    for di in range(5):
                        m1[dj, n, di * 28 + jj] = 1.0
    return e1t, c1, m1


def _conv2_tabs():
    e2t = np.zeros((96, 128), np.float32)
    for p in range(2):
        for o2 in range(12):
            for j2p in range(4):
                e2t[p * 48 + o2 * 4 + j2p, o2] = 1.0
    c2 = np.zeros((5, 360, 150), np.float32)
    m2 = np.zeros((5, 96, 360), np.float32)
    for dj in range(5):
        for di in range(5):
            for c in range(6):
                for ww in range(12):
                    c2[dj, di * 72 + c * 12 + ww, (di * 5 + dj) * 6 + c] = 1.0
        for p in range(2):
            for o2 in range(12):
                for j2p in range(4):
                    n = p * 48 + o2 * 4 + j2p
                    ww = 2 * j2p + p + dj
                    for di in range(5):
                        for c in range(6):
                            m2[dj, n, di * 72 + c * 12 + ww] = 1.0
    return e2t, c2, m2


def _fc1_tab():
    pf = np.zeros((192, 192), np.float32)
    for h in range(4):
        for c in range(12):
            for w in range(4):
                pf[h * 48 + c * 4 + w, h * 48 + w * 12 + c] = 1.0
    return pf


_E1T, _C1, _M1 = _conv1_tabs()
_E2T, _C2, _M2 = _conv2_tabs()
_PF = _fc1_tab()
_I128 = np.eye(128, dtype=np.float32)


def _dgt(a, b):
    return jax.lax.dot_general(a, b, (((1,), (1,)), ((), ())),
                               preferred_element_type=jnp.float32)


def _dot(a, b):
    return jnp.dot(a, b, preferred_element_type=jnp.float32)


def _dga(a, b):
    return jax.lax.dot_general(a, b, (((0,), (0,)), ((), ())),
                               preferred_element_type=jnp.float32)


def _lenet_kernel(x_ref, c1w, c1b, c2w, c2b, fc1w, fc2w, outw,
                  fc1b, fc2b, outb,
                  e1t, c1t, m1t, e2t, c2t, m2t, pf, i128,
                  o_ref, s1, s2):
    v1t = _dgt(e1t[...], c1w[...])
    w1f = jnp.zeros((144, 140), jnp.float32)
    for dj in range(5):
        w1f = w1f + _dgt(v1t, c1t[dj]) * m1t[dj]
    w1 = jnp.concatenate(
        [jnp.pad(w1f, ((0, 0), (0, 28))), jnp.pad(w1f, ((0, 0), (28, 0)))],
        axis=0).astype(OP_DT)
    b1 = _dgt(e1t[...], c1b[...])[:72]

    v2t = _dgt(e2t[...], c2w[...])
    w2f = jnp.zeros((96, 360), jnp.float32)
    for dj in range(5):
        w2f = w2f + _dgt(v2t, c2t[dj]) * m2t[dj]
    w2 = jnp.concatenate(
        [jnp.pad(w2f, ((0, 0), (0, 72))), jnp.pad(w2f, ((0, 0), (72, 0)))],
        axis=0).astype(OP_DT)
    b2 = _dgt(e2t[...], c2b[...])[:48]

    wf = jax.lax.dot_general(
        fc1w[...], pf[...], (((0,), (1,)), ((), ())),
        preferred_element_type=jnp.float32).astype(OP_DT)
    bf = _dgt(i128[...], fc1b[...])
    wg = fc2w[...].astype(OP_DT)
    bg = _dgt(i128[...], fc2b[...])
    wo = outw[...].astype(OP_DT)
    bo = _dgt(i128[...], outb[...])

    xb = x_ref[...].astype(OP_DT).reshape(784, -1)
    for r in range(12):
        d = _dot(w1, xb[r * 56:r * 56 + 168])
        m = jnp.maximum(jnp.maximum(d[:72], d[72:144]),
                        jnp.maximum(d[144:216], d[216:288]))
        s1[r * 72:(r + 1) * 72, :] = (
            jnp.maximum(m + b1, 0.0).astype(OP_DT))

    for r in range(4):
        d = _dot(w2, s1[r * 144:r * 144 + 432, :])
        m = jnp.maximum(jnp.maximum(d[:48], d[48:96]),
                        jnp.maximum(d[96:144], d[144:192]))
        s2[r * 48:(r + 1) * 48, :] = (
            jnp.maximum(m + b2, 0.0).astype(OP_DT))

    h = jnp.maximum(_dot(wf, s2[...]) + bf, 0.0)
    h = h.astype(OP_DT)
    h = jnp.maximum(_dga(wg, h) + bg, 0.0)
    h = h.astype(OP_DT)
    o = _dga(wo, h) + bo
    o_ref[...] = o[:10, :]


def kernel(c1_w, c1_b, c2_w, c2_b, fc1_w, fc1_b, fc2_w, fc2_b, out_w, out_b, x):
    B = x.shape[0]
    bl = BL if B % BL == 0 else B
    xt = jnp.transpose(x, (2, 3, 1, 0)).reshape(784, B // 128, 128)

    def _c(shape):
        return pl.BlockSpec(shape, lambda i: (0,) * len(shape))

    out_t = pl.pallas_call(
        _lenet_kernel,
        out_shape=jax.ShapeDtypeStruct((10, B), jnp.float32),
        grid=(B // bl,),
        in_specs=[pl.BlockSpec((784, bl // 128, 128), lambda i: (0, i, 0)),
                  _c((25, 128)), _c((1, 128)),
                  _c((150, 128)), _c((1, 128)),
                  _c((192, 128)), _c((128, 128)), _c((128, 128)),
                  _c((1, 128)), _c((1, 128)), _c((1, 128)),
                  _c((144, 128)), _c((5, 140, 25)), _c((5, 144, 140)),
                  _c((96, 128)), _c((5, 360, 150)), _c((5, 96, 360)),
                  _c((192, 192)), _c((128, 128))],
        out_specs=pl.BlockSpec((10, bl), lambda i: (0, i)),
        scratch_shapes=[pltpu.VMEM((12 * 72, bl), OP_DT),
                        pltpu.VMEM((4 * 48, bl), OP_DT)],
        compiler_params=pltpu.CompilerParams(
            dimension_semantics=("parallel",)),
    )(xt, c1_w, c1_b, c2_w, c2_b, fc1_w, fc2_w, out_w, fc1_b, fc2_b, out_b,
      _E1T, _C1, _M1, _E2T, _C2, _M2, _PF, _I128)
    return out_t.T
